# baseline (speedup 1.0000x reference)
import functools

import jax
import jax.numpy as jnp
from jax import lax
from jax.experimental import pallas as pl
from jax.experimental.pallas import tpu as pltpu
from jax.experimental.pallas import tpu_sc as plsc

B = 16384
EMB = 64
NF = 26
FEMB = 32
FEAT_VOCAB = 1000
NUM_ITEMS = 100000

NW = 32
PER_W = B // NW
C = 16
CHUNKS = PER_W // C
CF = C * NF
IDX_W = 104
NDMA = CF // IDX_W

_mesh = plsc.VectorSubcoreMesh(core_axis_name="c", subcore_axis_name="s")


@functools.partial(
    pl.kernel,
    out_type=jax.ShapeDtypeStruct((B,), jnp.float32),
    mesh=_mesh,
    compiler_params=pltpu.CompilerParams(use_tc_tiling_on_sc=True),
    scratch_types=[
        pltpu.VMEM((C,), jnp.int32),
        pltpu.VMEM((C,), jnp.int32),
        pltpu.VMEM((C,), jnp.int32),
        pltpu.VMEM((C,), jnp.int32),
        pltpu.VMEM((CF,), jnp.int32),
        pltpu.VMEM((CF,), jnp.int32),
        pltpu.VMEM((CF,), jnp.int32),
        pltpu.VMEM((C, 128), jnp.float32),
        pltpu.VMEM((C, 128), jnp.float32),
        pltpu.VMEM((C,), jnp.float32),
        pltpu.VMEM((C,), jnp.float32),
        pltpu.VMEM((CF, 128), jnp.float32),
        pltpu.VMEM((CF, 128), jnp.float32),
        pltpu.VMEM((C,), jnp.float32),
        pltpu.VMEM((8, 256), jnp.float32),
        pltpu.VMEM((16,), jnp.float32),
        pltpu.SemaphoreType.DMA,
    ],
)
def _mf_sc(u_id, i_id, feats, user_emb, user_bias, item_emb, item_bias,
           fu_tab, fi_tab, mean, out_hbm,
           u_v, i_v, u2_v, i2_v, f_v, fu_idx, fi_idx,
           U_v, I_v, bu_v, bi_v, FU_v, FI_v, out_v, P_v, mean_v, sem):
    wid = lax.axis_index("s") * 2 + lax.axis_index("c")
    base0 = wid * PER_W

    pltpu.sync_copy(mean, mean_v)
    lanes = lax.iota(jnp.int32, 16)

    def chunk(g, carry):
        base = base0 + g * C
        pltpu.sync_copy(u_id.at[pl.ds(base, C)], u_v)
        pltpu.sync_copy(i_id.at[pl.ds(base, C)], i_v)
        pltpu.sync_copy(feats.at[pl.ds(base * NF, CF)], f_v)

        u2_v[...] = lax.shift_right_logical(u_v[...], 1)
        i2_v[...] = lax.shift_right_logical(i_v[...], 1)
        ivec = lax.shift_right_logical(i_v[...], 2)
        for f in range(NF):
            s = pl.ds(f * C, 16)
            fu_idx[s] = f_v[s] + f * FEAT_VOCAB
            fi_idx[s] = ivec + f * (NUM_ITEMS * FEMB // 128)

        cps = [
            pltpu.async_copy(user_emb.at[u2_v], U_v, sem),
            pltpu.async_copy(item_emb.at[i2_v], I_v, sem),
            pltpu.async_copy(user_bias.at[u_v], bu_v, sem),
            pltpu.async_copy(item_bias.at[i_v], bi_v, sem),
        ]
        for j in range(NDMA):
            d = pl.ds(j * IDX_W, IDX_W)
            cps.append(pltpu.async_copy(fu_tab.at[fu_idx.at[d]], FU_v.at[d], sem))
            cps.append(pltpu.async_copy(fi_tab.at[fi_idx.at[d]], FI_v.at[d], sem))
        for cp in cps:
            cp.wait()

        def elt(b, _):
            for pu in range(2):
                for pi in range(2):
                    acc = (U_v[b, pl.ds(pu * 64, 16)]
                           * I_v[b, pl.ds(pi * 64, 16)])
                    for k in range(1, EMB // 16):
                        acc = acc + (U_v[b, pl.ds(pu * 64 + k * 16, 16)]
                                     * I_v[b, pl.ds(pi * 64 + k * 16, 16)])
                    _fold_store(P_v, pu * 2 + pi, b, acc)
            for q in range(4):
                acc = FU_v[b, pl.ds(0, 16)] * FI_v[b, pl.ds(q * 32, 16)]
                acc = acc + (FU_v[b, pl.ds(16, 16)]
                             * FI_v[b, pl.ds(q * 32 + 16, 16)])
                for f in range(1, NF):
                    r = f * C + b
                    for h in range(FEMB // 16):
                        acc = acc + (FU_v[r, pl.ds(h * 16, 16)]
                                     * FI_v[r, pl.ds(q * 32 + h * 16, 16)])
                _fold_store(P_v, 4 + q, b, acc)
            return _

        lax.fori_loop(0, C, elt, 0, unroll=False)

        ev = [None] * 8
        for v in range(8):
            for rnd, (w, n) in enumerate([(8, 8), (4, 4), (2, 2), (1, 1)]):
                for k in range(n):
                    a0 = 32 * k
                    t1 = P_v[v, pl.ds(a0, 16)] + P_v[v, pl.ds(a0 + w, 16)]
                    t2 = (P_v[v, pl.ds(a0 + 16 - w, 16)]
                          + P_v[v, pl.ds(a0 + 16, 16)])
                    sel = (lanes & (2 * w - 1)) < w
                    q_ = jnp.where(sel, t1, t2)
                    if rnd < 3:
                        P_v[v, pl.ds(16 * k, 16)] = q_
            ev[v] = q_

        mu = (u_v[...] & 1) == 0
        mi = (i_v[...] & 1) == 0
        ui = jnp.where(mu, jnp.where(mi, ev[0], ev[1]),
                       jnp.where(mi, ev[2], ev[3]))
        iq = i_v[...] & 3
        fsum = jnp.where(iq == 0, ev[4],
                         jnp.where(iq == 1, ev[5],
                                   jnp.where(iq == 2, ev[6], ev[7])))
        s16 = pl.ds(0, 16)
        out_v[s16] = ui + fsum + bu_v[s16] + bi_v[s16] + mean_v[s16]

        pltpu.sync_copy(out_v, out_hbm.at[pl.ds(base, C)])
        return carry

    lax.fori_loop(0, CHUNKS, chunk, 0, unroll=False)


def _fold_store(P_v, v, b, acc):
    br = ((b & 1) << 3) | ((b & 2) << 1) | ((b & 4) >> 1) | ((b & 8) >> 3)
    P_v[v, pl.ds(br * 16, 16)] = acc


def kernel(u_id, i_id, features, user_emb, user_bias, item_emb, item_bias,
           feat_u, feat_i, mean):
    u_id = u_id.astype(jnp.int32)
    i_id = i_id.astype(jnp.int32)
    feats = (features.astype(jnp.int32)
             .reshape(B // C, C, NF).transpose(0, 2, 1).reshape(-1))
    ue = user_emb.reshape(-1, 128)
    ie = item_emb.reshape(-1, 128)
    fi_tab = feat_i.reshape(-1, 128)
    fu_tab = jnp.pad(feat_u.reshape(NF * FEAT_VOCAB, FEMB),
                     ((0, 0), (0, 128 - FEMB)))
    ub = user_bias.reshape(-1)
    ib = item_bias.reshape(-1)
    mean16 = jnp.broadcast_to(mean, (16,))
    return _mf_sc(u_id, i_id, feats, ue, ub, ie,
                  ib, fu_tab, fi_tab, mean16)
